# Initial kernel scaffold; baseline (speedup 1.0000x reference)
import jax
import jax.numpy as jnp
from jax.experimental import pallas as pl
from jax.experimental.pallas import tpu as pltpu

G = 316
NN = G * G
HID = 64
C = 79 * 128
T = 10
NPAD = T * C
WLEN = C + 640
LPAD = 101760


def _mlp_kernel(x_ref, wembT_ref, bembT_ref, whidT_ref, bhidT_ref,
                wpost_ref, bpost_ref, out_ref, acc_ref):
    t = pl.program_id(1)

    @pl.when(t == 0)
    def _():
        acc_ref[...] = jnp.zeros_like(acc_ref)

    base = t * C
    w0 = x_ref[0, 0, pl.ds(base, WLEN)]
    w1 = x_ref[0, 1, pl.ds(base, WLEN)]

    def sh(w, off):
        return jax.lax.slice_in_dim(w, 316 + off, 316 + off + C).reshape(1, C)

    x0 = sh(w0, 0)
    x1 = sh(w1, 0)
    s_up = x0 * sh(w0, 316) + x1 * sh(w1, 316)
    s_down = x0 * sh(w0, -316) + x1 * sh(w1, -316)
    p_m1 = x0 * sh(w0, -1) + x1 * sh(w1, -1)
    p_p1 = x0 * sh(w0, 1) + x1 * sh(w1, 1)
    p_p315 = x0 * sh(w0, 315) + x1 * sh(w1, 315)
    p_m315 = x0 * sh(w0, -315) + x1 * sh(w1, -315)

    gidx = base + jax.lax.broadcasted_iota(jnp.int32, (1, C), 1)
    xx = gidx % G
    s_left = jnp.where(xx == 0, p_p315, p_m1)
    s_right = jnp.where(xx == G - 1, p_m315, p_p1)

    S = jnp.concatenate([s_up, s_down, s_left, s_right], axis=0)
    h1 = jnp.dot(wembT_ref[...], S, preferred_element_type=jnp.float32)
    h1 = jnp.maximum(h1 + bembT_ref[...], 0.0)
    h2 = jnp.dot(whidT_ref[...], h1, preferred_element_type=jnp.float32)
    h2 = jnp.maximum(h2 + bhidT_ref[...], 0.0)

    valid = (gidx < NN).astype(jnp.float32)
    acc_ref[:, :1] += jnp.sum(h2 * valid, axis=1, keepdims=True)

    @pl.when(t == T - 1)
    def _():
        res = jnp.sum(acc_ref[:, :1] * wpost_ref[...]) + NN * bpost_ref[0, 0]
        out_ref[...] = jnp.full((1, 128), res, dtype=jnp.float32)


@jax.jit
def kernel(x, W_emb, b_emb, W_hid, b_hid, W_post, b_post):
    B = x.shape[0]
    xT = jnp.transpose(x, (0, 2, 1))
    pad = jnp.zeros((B, 2, LPAD - (NN + 2 * G)), jnp.float32)
    xh = jnp.concatenate([xT[:, :, NN - G:], xT, xT[:, :, :G], pad], axis=2)

    out = pl.pallas_call(
        _mlp_kernel,
        grid=(B, T),
        in_specs=[
            pl.BlockSpec((1, 2, LPAD), lambda b, t: (b, 0, 0)),
            pl.BlockSpec((HID, 4), lambda b, t: (0, 0)),
            pl.BlockSpec((HID, 1), lambda b, t: (0, 0)),
            pl.BlockSpec((HID, HID), lambda b, t: (0, 0)),
            pl.BlockSpec((HID, 1), lambda b, t: (0, 0)),
            pl.BlockSpec((HID, 1), lambda b, t: (0, 0)),
            pl.BlockSpec((1, 1), lambda b, t: (0, 0)),
        ],
        out_specs=pl.BlockSpec((1, 128), lambda b, t: (b, 0)),
        out_shape=jax.ShapeDtypeStruct((B, 128), jnp.float32),
        scratch_shapes=[pltpu.VMEM((HID, 128), jnp.float32)],
    )(xh, W_emb.T, b_emb[:, None], W_hid.T, b_hid[:, None],
      W_post, b_post.reshape(1, 1))
    return out[:, :1]


# speedup vs baseline: 8.7896x; 8.7896x over previous
import jax
import jax.numpy as jnp
from jax.experimental import pallas as pl
from jax.experimental.pallas import tpu as pltpu

G = 316
NN = G * G
HID = 64
C = 79 * 128
T = 10
NPAD = T * C
WLEN = C + 640
LPAD = 101760


def _mlp_kernel(x_ref, wembT_ref, bembT_ref, whidT_ref, bhidT_ref,
                wpost_ref, bpost_ref, out_ref, acc_ref):
    t = pl.program_id(1)

    @pl.when(t == 0)
    def _():
        acc_ref[...] = jnp.zeros_like(acc_ref)

    base = t * C
    w0 = x_ref[0, 0, pl.ds(base, WLEN)]
    w1 = x_ref[0, 1, pl.ds(base, WLEN)]

    def sh(w, off):
        return jax.lax.slice_in_dim(w, 316 + off, 316 + off + C).reshape(1, C)

    x0 = sh(w0, 0)
    x1 = sh(w1, 0)
    s_up = x0 * sh(w0, 316) + x1 * sh(w1, 316)
    s_down = x0 * sh(w0, -316) + x1 * sh(w1, -316)
    p_m1 = x0 * sh(w0, -1) + x1 * sh(w1, -1)
    p_p1 = x0 * sh(w0, 1) + x1 * sh(w1, 1)
    p_p315 = x0 * sh(w0, 315) + x1 * sh(w1, 315)
    p_m315 = x0 * sh(w0, -315) + x1 * sh(w1, -315)

    gidx = base + jax.lax.broadcasted_iota(jnp.int32, (1, C), 1)
    xx = gidx % G
    s_left = jnp.where(xx == 0, p_p315, p_m1)
    s_right = jnp.where(xx == G - 1, p_m315, p_p1)

    S = jnp.concatenate([s_up, s_down, s_left, s_right], axis=0)
    h1 = jnp.dot(wembT_ref[...], S, preferred_element_type=jnp.float32)
    h1 = jnp.maximum(h1 + bembT_ref[...], 0.0)
    h2 = jnp.dot(whidT_ref[...], h1, preferred_element_type=jnp.float32)
    h2 = jnp.maximum(h2 + bhidT_ref[...], 0.0)

    valid = (gidx < NN).astype(jnp.float32)
    acc_ref[:, :1] += jnp.sum(h2 * valid, axis=1, keepdims=True)

    @pl.when(t == T - 1)
    def _():
        res = jnp.sum(acc_ref[:, :1] * wpost_ref[...]) + NN * bpost_ref[0, 0]
        out_ref[...] = jnp.full((1, 8, 128), res, dtype=jnp.float32)


@jax.jit
def kernel(x, W_emb, b_emb, W_hid, b_hid, W_post, b_post):
    B = x.shape[0]
    xT = jnp.transpose(x, (0, 2, 1))
    pad = jnp.zeros((B, 2, LPAD - (NN + 2 * G)), jnp.float32)
    xh = jnp.concatenate([xT[:, :, NN - G:], xT, xT[:, :, :G], pad], axis=2)

    out = pl.pallas_call(
        _mlp_kernel,
        grid=(B, T),
        in_specs=[
            pl.BlockSpec((1, 2, LPAD), lambda b, t: (b, 0, 0)),
            pl.BlockSpec((HID, 4), lambda b, t: (0, 0)),
            pl.BlockSpec((HID, 1), lambda b, t: (0, 0)),
            pl.BlockSpec((HID, HID), lambda b, t: (0, 0)),
            pl.BlockSpec((HID, 1), lambda b, t: (0, 0)),
            pl.BlockSpec((HID, 1), lambda b, t: (0, 0)),
            pl.BlockSpec((1, 1), lambda b, t: (0, 0)),
        ],
        out_specs=pl.BlockSpec((1, 8, 128), lambda b, t: (b, 0, 0)),
        out_shape=jax.ShapeDtypeStruct((B, 8, 128), jnp.float32),
        scratch_shapes=[pltpu.VMEM((HID, 128), jnp.float32)],
    )(xh, W_emb.T, b_emb[:, None], W_hid.T, b_hid[:, None],
      W_post, b_post.reshape(1, 1))
    return out[:, 0, :1]
